# baseline (speedup 1.0000x reference)
import jax
import jax.numpy as jnp
from jax.experimental import pallas as pl
from jax.experimental.pallas import tpu as pltpu

_H = 32
_W = 32
_D = 256
_B = 8
_C = 2 * _D
_G = 2
_BPG = _B // _G


def _tc_body(row_ref, col_ref, out_ref, slab, sem):
    g = pl.program_id(0)
    col = col_ref[0:_W, :]
    row = row_ref[0:_H, :]
    slab[:, :, 0:_D] = jnp.broadcast_to(col[None, :, :], (_H, _W, _D))
    slab[:, :, _D:_C] = jnp.broadcast_to(row[:, None, :], (_H, _W, _D))
    copies = [pltpu.make_async_copy(slab, out_ref.at[g * _BPG + b], sem)
              for b in range(_BPG)]
    for cp in copies:
        cp.start()
    for cp in copies:
        cp.wait()


@jax.jit
def _pos_embed(row_embed, col_embed):
    out = pl.pallas_call(
        _tc_body,
        grid=(_G,),
        out_shape=jax.ShapeDtypeStruct((_B, _H, _W, _C), jnp.float32),
        in_specs=[
            pl.BlockSpec(memory_space=pltpu.VMEM),
            pl.BlockSpec(memory_space=pltpu.VMEM),
        ],
        out_specs=pl.BlockSpec(memory_space=pl.ANY),
        scratch_shapes=[
            pltpu.VMEM((_H, _W, _C), jnp.float32),
            pltpu.SemaphoreType.DMA,
        ],
        compiler_params=pltpu.CompilerParams(
            dimension_semantics=("parallel",)),
    )(row_embed, col_embed)
    return jnp.transpose(out, (0, 3, 1, 2))


def kernel(x, row_embed, col_embed):
    assert x.shape[0] == _B and x.shape[-2:] == (_H, _W)
    return _pos_embed(row_embed, col_embed)


# speedup vs baseline: 1.1835x; 1.1835x over previous
import jax
import jax.numpy as jnp
from jax.experimental import pallas as pl
from jax.experimental.pallas import tpu as pltpu

_H = 32
_W = 32
_D = 256
_B = 8
_C = 2 * _D
_RC = 8
_NCH = _H // _RC


def _tc_body(row_ref, col_ref, out_ref, slab, sem):
    col = col_ref[0:_W, :]
    copies = []
    for ci in range(_NCH):
        i0 = ci * _RC
        rows = row_ref[i0:i0 + _RC, :]
        slab[pl.ds(i0, _RC), :, 0:_D] = jnp.broadcast_to(
            col[None, :, :], (_RC, _W, _D))
        slab[pl.ds(i0, _RC), :, _D:_C] = jnp.broadcast_to(
            rows[:, None, :], (_RC, _W, _D))
        chunk = slab.at[pl.ds(i0, _RC)]
        for b in range(_B):
            cp = pltpu.make_async_copy(
                chunk, out_ref.at[b, pl.ds(i0, _RC)], sem)
            cp.start()
            copies.append(cp)
    for cp in copies:
        cp.wait()


@jax.jit
def _pos_embed(row_embed, col_embed):
    out = pl.pallas_call(
        _tc_body,
        out_shape=jax.ShapeDtypeStruct((_B, _H, _W, _C), jnp.float32),
        in_specs=[
            pl.BlockSpec(memory_space=pltpu.VMEM),
            pl.BlockSpec(memory_space=pltpu.VMEM),
        ],
        out_specs=pl.BlockSpec(memory_space=pl.ANY),
        scratch_shapes=[
            pltpu.VMEM((_H, _W, _C), jnp.float32),
            pltpu.SemaphoreType.DMA,
        ],
    )(row_embed, col_embed)
    return jnp.transpose(out, (0, 3, 1, 2))


def kernel(x, row_embed, col_embed):
    assert x.shape[0] == _B and x.shape[-2:] == (_H, _W)
    return _pos_embed(row_embed, col_embed)
